# baseline (speedup 1.0000x reference)
import functools

import numpy as np
import jax
import jax.numpy as jnp
from jax import lax
from jax.experimental import pallas as pl
from jax.experimental.pallas import tpu as pltpu
from jax.experimental.pallas import tpu_sc as plsc

_D = 768
_PE_LEN = 2048
_BATCH = 4
_SCALE = float(np.sqrt(float(_D)))

_NC = 2
_NS = 16
_NW = _NC * _NS
_B = _BATCH * _PE_LEN
_ROWS_PER_W = _B // _NW
_SUB = 64
_NSUB = _ROWS_PER_W // _SUB
_LANES = 16
_VPR = _D // _LANES


def _pe_table() -> np.ndarray:
    half = _D / 2
    positions = np.arange(_PE_LEN)[:, np.newaxis]
    depths = np.arange(half)[np.newaxis, :] / half
    angle_rates = 1.0 / (10000.0 ** depths)
    angle_rads = positions * angle_rates
    pe = np.concatenate([np.sin(angle_rads), np.cos(angle_rads)], axis=-1)
    return pe.astype(np.float32)


_PE_CONST = _pe_table()

_mesh = plsc.VectorSubcoreMesh(core_axis_name="c", subcore_axis_name="s")


@functools.partial(
    pl.kernel,
    out_type=jax.ShapeDtypeStruct((_B, _D), jnp.float32),
    mesh=_mesh,
    scratch_types=[
        pltpu.VMEM((_ROWS_PER_W,), jnp.int32),
        pltpu.VMEM((_SUB, _D), jnp.float32),
        pltpu.VMEM((_SUB, _D), jnp.float32),
        pltpu.SemaphoreType.DMA,
    ],
)
def _sc_embed(x_hbm, table_hbm, pe_hbm, out_hbm, idx_v, pe_v, em_v, sem):
    wid = lax.axis_index("s") * _NC + lax.axis_index("c")
    base = wid * _ROWS_PER_W
    pe_base = lax.rem(base, _PE_LEN)
    pltpu.sync_copy(x_hbm.at[pl.ds(base, _ROWS_PER_W)], idx_v)
    for k in range(_NSUB):
        pltpu.sync_copy(pe_hbm.at[pl.ds(pe_base + k * _SUB, _SUB)], pe_v)
        pltpu.async_copy(
            table_hbm.at[idx_v.at[pl.ds(k * _SUB, _SUB)]], em_v, sem
        ).wait()

        @pl.loop(0, _SUB)
        def _fma_row(r):
            for j in range(_VPR):
                sl = pl.ds(j * _LANES, _LANES)
                em_v[r, sl] = em_v[r, sl] * _SCALE + pe_v[r, sl]

        pltpu.sync_copy(em_v, out_hbm.at[pl.ds(base + k * _SUB, _SUB)])


def kernel(x, table):
    pe = jnp.asarray(_PE_CONST)
    xf = x.reshape(-1).astype(jnp.int32)
    out = _sc_embed(xf, table, pe)
    return out.reshape(_BATCH, _PE_LEN, _D)


# speedup vs baseline: 1.2464x; 1.2464x over previous
import functools

import numpy as np
import jax
import jax.numpy as jnp
from jax import lax
from jax.experimental import pallas as pl
from jax.experimental.pallas import tpu as pltpu
from jax.experimental.pallas import tpu_sc as plsc

_D = 768
_PE_LEN = 2048
_BATCH = 4
_SCALE = float(np.sqrt(float(_D)))

_NC = 2
_NS = 16
_NW = _NC * _NS
_B = _BATCH * _PE_LEN
_ROWS_PER_W = _B // _NW
_SUB = 32
_NSUB = _ROWS_PER_W // _SUB
_LANES = 16
_VPR = _D // _LANES
_NBUF = 2


def _pe_table() -> np.ndarray:
    half = _D / 2
    positions = np.arange(_PE_LEN)[:, np.newaxis]
    depths = np.arange(half)[np.newaxis, :] / half
    angle_rates = 1.0 / (10000.0 ** depths)
    angle_rads = positions * angle_rates
    pe = np.concatenate([np.sin(angle_rads), np.cos(angle_rads)], axis=-1)
    return pe.astype(np.float32)


_PE_CONST = _pe_table()

_mesh = plsc.VectorSubcoreMesh(core_axis_name="c", subcore_axis_name="s")


@functools.partial(
    pl.kernel,
    out_type=jax.ShapeDtypeStruct((_B, _D), jnp.float32),
    mesh=_mesh,
    scratch_types=[
        pltpu.VMEM((_ROWS_PER_W,), jnp.int32),
        [pltpu.VMEM((_SUB, _D), jnp.float32) for _ in range(_NBUF)],
        [pltpu.VMEM((_SUB, _D), jnp.float32) for _ in range(_NBUF)],
        [pltpu.SemaphoreType.DMA for _ in range(_NBUF)],
        [pltpu.SemaphoreType.DMA for _ in range(_NBUF)],
        [pltpu.SemaphoreType.DMA for _ in range(_NBUF)],
    ],
)
def _sc_embed(x_hbm, table_hbm, pe_hbm, out_hbm,
              idx_v, pe_v, em_v, sem_g, sem_p, sem_o):
    wid = lax.axis_index("s") * _NC + lax.axis_index("c")
    base = wid * _ROWS_PER_W
    pe_base = lax.rem(base, _PE_LEN)
    pltpu.sync_copy(x_hbm.at[pl.ds(base, _ROWS_PER_W)], idx_v)

    def start_in(k):
        s = k % _NBUF
        pltpu.async_copy(
            pe_hbm.at[pl.ds(pe_base + k * _SUB, _SUB)], pe_v[s], sem_p[s]
        )
        pltpu.async_copy(
            table_hbm.at[idx_v.at[pl.ds(k * _SUB, _SUB)]], em_v[s], sem_g[s]
        )

    start_in(0)
    for k in range(_NSUB):
        s = k % _NBUF
        if k + 1 < _NSUB:
            if k + 1 >= _NBUF:
                pltpu.make_async_copy(
                    em_v[(k + 1) % _NBUF],
                    out_hbm.at[pl.ds(base + (k + 1 - _NBUF) * _SUB, _SUB)],
                    sem_o[(k + 1) % _NBUF],
                ).wait()
            start_in(k + 1)
        pltpu.make_async_copy(
            pe_hbm.at[pl.ds(pe_base + k * _SUB, _SUB)], pe_v[s], sem_p[s]
        ).wait()
        pltpu.make_async_copy(
            table_hbm.at[idx_v.at[pl.ds(k * _SUB, _SUB)]], em_v[s], sem_g[s]
        ).wait()

        @pl.loop(0, _SUB)
        def _fma_row(r):
            for j in range(_VPR):
                sl = pl.ds(j * _LANES, _LANES)
                em_v[s][r, sl] = em_v[s][r, sl] * _SCALE + pe_v[s][r, sl]

        pltpu.async_copy(
            em_v[s], out_hbm.at[pl.ds(base + k * _SUB, _SUB)], sem_o[s]
        )

    for k in range(_NSUB - _NBUF, _NSUB):
        s = k % _NBUF
        pltpu.make_async_copy(
            em_v[s], out_hbm.at[pl.ds(base + k * _SUB, _SUB)], sem_o[s]
        ).wait()


def kernel(x, table):
    pe = jnp.asarray(_PE_CONST)
    xf = x.reshape(-1).astype(jnp.int32)
    out = _sc_embed(xf, table, pe)
    return out.reshape(_BATCH, _PE_LEN, _D)


# speedup vs baseline: 1.3211x; 1.0599x over previous
import functools

import numpy as np
import jax
import jax.numpy as jnp
from jax import lax
from jax.experimental import pallas as pl
from jax.experimental.pallas import tpu as pltpu
from jax.experimental.pallas import tpu_sc as plsc

_D = 768
_PE_LEN = 2048
_BATCH = 4
_SCALE = float(np.sqrt(float(_D)))

_NC = 2
_NS = 16
_NW = _NC * _NS
_B = _BATCH * _PE_LEN
_ROWS_PER_W = _B // _NW
_SUB = 32
_NSUB = _ROWS_PER_W // _SUB
_LANES = 16
_VPR = _D // _LANES
_NBUF = 2
_PE_BLOCKS = 4
_PE_SH_ROWS = _PE_BLOCKS * 256
_ST_ROWS = _PE_SH_ROWS // _NS


def _pe_table() -> np.ndarray:
    half = _D / 2
    positions = np.arange(_PE_LEN)[:, np.newaxis]
    depths = np.arange(half)[np.newaxis, :] / half
    angle_rates = 1.0 / (10000.0 ** depths)
    angle_rads = positions * angle_rates
    pe = np.concatenate([np.sin(angle_rads), np.cos(angle_rads)], axis=-1)
    return pe.astype(np.float32)


_PE_CONST = _pe_table()

_mesh = plsc.VectorSubcoreMesh(core_axis_name="c", subcore_axis_name="s")


@functools.partial(
    pl.kernel,
    out_type=jax.ShapeDtypeStruct((_B, _D), jnp.float32),
    mesh=_mesh,
    scratch_types=[
        pltpu.VMEM((_ROWS_PER_W,), jnp.int32),
        pltpu.VMEM((_SUB, _D), jnp.float32),
        [pltpu.VMEM((_SUB, _D), jnp.float32) for _ in range(_NBUF)],
        pltpu.VMEM_SHARED((_PE_SH_ROWS, _D), jnp.float32),
        [pltpu.SemaphoreType.DMA for _ in range(_NBUF)],
        pltpu.SemaphoreType.DMA,
        [pltpu.SemaphoreType.DMA for _ in range(_NBUF)],
        pltpu.SemaphoreType.DMA,
    ],
)
def _sc_embed(x_hbm, table_hbm, pe_hbm, out_hbm,
              idx_v, pe_v, em_v, pe_sh, sem_g, sem_p, sem_o, sem_st):
    c = lax.axis_index("c")
    s = lax.axis_index("s")
    wid = s * _NC + c
    base = wid * _ROWS_PER_W
    blk = lax.rem(s, _PE_BLOCKS)
    pe_sh_base = blk * 256

    st_blk = lax.div(s, _PE_BLOCKS)
    st_hbm = (2 * st_blk + c) * 256 + lax.rem(s, _PE_BLOCKS) * _ST_ROWS
    pltpu.async_copy(
        pe_hbm.at[pl.ds(st_hbm, _ST_ROWS)],
        pe_sh.at[pl.ds(s * _ST_ROWS, _ST_ROWS)],
        sem_st,
    )

    pltpu.sync_copy(x_hbm.at[pl.ds(base, _ROWS_PER_W)], idx_v)

    def start_gather(k):
        b = k % _NBUF
        pltpu.async_copy(
            table_hbm.at[idx_v.at[pl.ds(k * _SUB, _SUB)]], em_v[b], sem_g[b]
        )

    def start_pe(k):
        pltpu.async_copy(
            pe_sh.at[pl.ds(pe_sh_base + k * _SUB, _SUB)], pe_v, sem_p
        )

    start_gather(0)
    pltpu.make_async_copy(
        pe_hbm.at[pl.ds(st_hbm, _ST_ROWS)],
        pe_sh.at[pl.ds(s * _ST_ROWS, _ST_ROWS)],
        sem_st,
    ).wait()
    plsc.subcore_barrier()
    start_pe(0)

    for k in range(_NSUB):
        b = k % _NBUF
        if k + 1 < _NSUB:
            if k + 1 >= _NBUF:
                pltpu.make_async_copy(
                    em_v[(k + 1) % _NBUF],
                    out_hbm.at[pl.ds(base + (k + 1 - _NBUF) * _SUB, _SUB)],
                    sem_o[(k + 1) % _NBUF],
                ).wait()
            start_gather(k + 1)
        pltpu.make_async_copy(
            pe_sh.at[pl.ds(pe_sh_base + k * _SUB, _SUB)], pe_v, sem_p
        ).wait()
        pltpu.make_async_copy(
            table_hbm.at[idx_v.at[pl.ds(k * _SUB, _SUB)]], em_v[b], sem_g[b]
        ).wait()

        @pl.loop(0, _SUB)
        def _fma_row(r):
            erow = em_v[b].at[r]
            prow = pe_v.at[r]
            for j in range(_VPR):
                sl = pl.ds(j * _LANES, _LANES)
                erow[sl] = erow[sl] * _SCALE + prow[sl]

        if k + 1 < _NSUB:
            start_pe(k + 1)
        pltpu.async_copy(
            em_v[b], out_hbm.at[pl.ds(base + k * _SUB, _SUB)], sem_o[b]
        )

    for k in range(_NSUB - _NBUF, _NSUB):
        b = k % _NBUF
        pltpu.make_async_copy(
            em_v[b], out_hbm.at[pl.ds(base + k * _SUB, _SUB)], sem_o[b]
        ).wait()


def kernel(x, table):
    pe = jnp.asarray(_PE_CONST)
    xf = x.reshape(-1).astype(jnp.int32)
    out = _sc_embed(xf, table, pe)
    return out.reshape(_BATCH, _PE_LEN, _D)
